# baseline (speedup 1.0000x reference)
import jax
import jax.numpy as jnp
from jax import lax
from jax.experimental import pallas as pl
from jax.experimental.pallas import tpu as pltpu
from jax.experimental.pallas import tpu_sc as plsc

D_VOCAB = 100000
SPLIT = 49920
LEN0 = SPLIT
LEN1 = D_VOCAB - SPLIT
N_ROWS = 1024
NC = 2
NS = 16
L = 16
NW = NC * NS
ROWS_PER_W = N_ROWS // NW


def _body(x_hbm, out_hbm, idx_v, buf0, buf1, sem0, sem1):
    wid = lax.axis_index("s") * NC + lax.axis_index("c")
    base = wid * ROWS_PER_W

    pltpu.sync_copy(x_hbm.at[pl.ds(base * 2, 2 * ROWS_PER_W)], idx_v)

    zeros16 = jnp.zeros((L,), jnp.float32)
    ones16 = jnp.ones((L,), jnp.float32)
    iota16 = lax.iota(jnp.int32, L)

    bufs = (buf0, buf1)
    sems = (sem0, sem1)
    offs = (0, SPLIT)
    lens = (LEN0, LEN1)

    def _zero(i, carry):
        buf0[pl.ds(i * L, L)] = zeros16
        buf1[pl.ds(i * L, L)] = zeros16
        return carry

    lax.fori_loop(0, LEN0 // L, _zero, 0)

    def _zero1(i, carry):
        buf1[pl.ds(LEN0 + i * L, L)] = zeros16
        return carry

    lax.fori_loop(0, (LEN1 - LEN0) // L, _zero1, 0)

    handles = [None, None]
    prev = [None, None]
    for r in range(ROWS_PER_W):
        c, k = divmod(r, 8)
        if k == 0:
            chunk = idx_v[pl.ds(c * L, L)]
            in0 = chunk < SPLIT
        m0 = iota16 == (2 * k)
        m1 = iota16 == (2 * k + 1)
        for h in range(2):
            inh = in0 if h == 0 else jnp.logical_not(in0)
            local = jnp.clip(chunk - offs[h], 0, lens[h] - 1)
            if handles[h] is not None:
                handles[h].wait()
                plsc.store_scatter(bufs[h], [prev[h][0]], zeros16,
                                   mask=prev[h][1])
            plsc.addupdate_scatter(bufs[h], [local], ones16, mask=m0 & inh)
            plsc.addupdate_scatter(bufs[h], [local], ones16, mask=m1 & inh)
            handles[h] = pltpu.async_copy(
                bufs[h],
                out_hbm.at[base + r, 0, pl.ds(offs[h], lens[h])],
                sems[h])
            prev[h] = (local, (m0 | m1) & inh)
    handles[0].wait()
    handles[1].wait()


@jax.jit
def _embed(x_flat):
    mesh = plsc.VectorSubcoreMesh(
        core_axis_name="c", subcore_axis_name="s", num_cores=NC,
        num_subcores=NS)
    f = pl.kernel(
        _body,
        out_type=jax.ShapeDtypeStruct((N_ROWS, 1, D_VOCAB), jnp.float32),
        mesh=mesh,
        scratch_types=[
            pltpu.VMEM((2 * ROWS_PER_W,), jnp.int32),
            pltpu.VMEM((LEN0,), jnp.float32),
            pltpu.VMEM((LEN1,), jnp.float32),
            pltpu.SemaphoreType.DMA,
            pltpu.SemaphoreType.DMA,
        ],
        compiler_params=pltpu.CompilerParams(needs_layout_passes=False),
    )
    return f(x_flat)


def kernel(x, W_E):
    del W_E
    return _embed(x.reshape(-1).astype(jnp.int32))


# speedup vs baseline: 2.3454x; 2.3454x over previous
import jax
import jax.numpy as jnp
from jax import lax
from jax.experimental import pallas as pl
from jax.experimental.pallas import tpu as pltpu
from jax.experimental.pallas import tpu_sc as plsc

D_VOCAB = 100000
SPLIT = 49920
LEN0 = SPLIT
LEN1 = D_VOCAB - SPLIT
N_ROWS = 1024
NC = 2
NS = 16
L = 16
NW = NC * NS
ROWS_PER_W = N_ROWS // NW


def _body(x_hbm, out_hbm, idx_v, buf0, buf1, sem0, sem1):
    wid = lax.axis_index("s") * NC + lax.axis_index("c")
    base = wid * ROWS_PER_W

    pltpu.sync_copy(x_hbm.at[pl.ds(base * 2, 2 * ROWS_PER_W)], idx_v)

    zeros16 = jnp.zeros((L,), jnp.float32)
    ones16 = jnp.ones((L,), jnp.float32)
    iota16 = lax.iota(jnp.int32, L)

    bufs = (buf0, buf1)
    sems = (sem0, sem1)
    offs = (0, SPLIT)
    lens = (LEN0, LEN1)

    def _zero(i, carry):
        buf0[pl.ds(i * L, L)] = zeros16
        buf1[pl.ds(i * L, L)] = zeros16
        return carry

    lax.fori_loop(0, LEN0 // L, _zero, 0)

    def _zero1(i, carry):
        buf1[pl.ds(LEN0 + i * L, L)] = zeros16
        return carry

    lax.fori_loop(0, (LEN1 - LEN0) // L, _zero1, 0)

    handles = [None, None]
    prev = [None, None]
    for r in range(ROWS_PER_W):
        c, k = divmod(r, 8)
        if k == 0:
            chunk = idx_v[pl.ds(c * L, L)]
            in0 = chunk < SPLIT
        m0 = iota16 == (2 * k)
        m1 = iota16 == (2 * k + 1)
        for h in range(2):
            inh = in0 if h == 0 else jnp.logical_not(in0)
            local = jnp.clip(chunk - offs[h], 0, lens[h] - 1)
            if handles[h] is not None:
                handles[h].wait()
                plsc.store_scatter(bufs[h], [prev[h][0]], zeros16,
                                   mask=prev[h][1])
            plsc.addupdate_scatter(bufs[h], [local], ones16, mask=m0 & inh)
            plsc.addupdate_scatter(bufs[h], [local], ones16, mask=m1 & inh)
            handles[h] = pltpu.async_copy(
                bufs[h],
                out_hbm.at[base + r].at[pl.ds(offs[h], lens[h])],
                sems[h])
            prev[h] = (local, (m0 | m1) & inh)
    handles[0].wait()
    handles[1].wait()


@jax.jit
def _embed(x_flat):
    mesh = plsc.VectorSubcoreMesh(
        core_axis_name="c", subcore_axis_name="s", num_cores=NC,
        num_subcores=NS)
    f = pl.kernel(
        _body,
        out_type=jax.ShapeDtypeStruct((N_ROWS, D_VOCAB), jnp.float32),
        mesh=mesh,
        scratch_types=[
            pltpu.VMEM((2 * ROWS_PER_W,), jnp.int32),
            pltpu.VMEM((LEN0,), jnp.float32),
            pltpu.VMEM((LEN1,), jnp.float32),
            pltpu.SemaphoreType.DMA,
            pltpu.SemaphoreType.DMA,
        ],
        compiler_params=pltpu.CompilerParams(needs_layout_passes=False),
    )
    return f(x_flat)


def kernel(x, W_E):
    del W_E
    return _embed(x.reshape(-1).astype(jnp.int32))[:, None, :]


# speedup vs baseline: 6.5136x; 2.7771x over previous
import jax
import jax.numpy as jnp
from jax import lax
from jax.experimental import pallas as pl
from jax.experimental.pallas import tpu as pltpu
from jax.experimental.pallas import tpu_sc as plsc

D_VOCAB = 100000
N_ROWS = 1024
N_TOK = 2 * N_ROWS
CH = 32
NCHUNK = D_VOCAB // CH
NC = 2
NS = 16
L = 16
NW = NC * NS
QFULL = NCHUNK // NW
NEXTRA = NCHUNK - QFULL * NW
LISTCAP = N_ROWS + L


def _body(x_hbm, out_hbm, idx_v, cidA, addrA, cidB, addrB, buf, sem):
    wid = lax.axis_index("s") * NC + lax.axis_index("c")

    pltpu.sync_copy(x_hbm, idx_v)

    zeros16 = jnp.zeros((L,), jnp.float32)
    ones16 = jnp.ones((L,), jnp.float32)
    iota16 = lax.iota(jnp.int32, L)
    even16 = (iota16 & 1) == 0
    half16 = iota16 >> 1

    def _zero(i, carry):
        buf[i >> 6, pl.ds((i & 63) * L, L)] = zeros16
        return carry

    lax.fori_loop(0, 64 * 64, _zero, 0)

    def _scan(q, carry):
        ca, cb = carry
        tvec = idx_v[pl.ds(q * L, L)]
        cid = tvec >> 5
        mine = (cid & (NW - 1)) == wid
        bvec = q * 8 + half16
        addr = ((tvec & (CH - 1)) << 10) | bvec
        mA = mine & even16
        mB = mine & jnp.logical_not(even16)
        plsc.store_compressed(cidA.at[pl.ds(ca, L)], cid, mask=mA)
        plsc.store_compressed(addrA.at[pl.ds(ca, L)], addr, mask=mA)
        plsc.store_compressed(cidB.at[pl.ds(cb, L)], cid, mask=mB)
        plsc.store_compressed(addrB.at[pl.ds(cb, L)], addr, mask=mB)
        ca = ca + jnp.sum(mA.astype(jnp.int32))
        cb = cb + jnp.sum(mB.astype(jnp.int32))
        return ca, cb

    cntA, cntB = lax.fori_loop(0, N_TOK // L, _scan, (jnp.int32(0),
                                                      jnp.int32(0)))
    nvA = (cntA + L - 1) >> 4
    nvB = (cntB + L - 1) >> 4

    def _pass(j, row_off, value):
        def _one(cid_ref, addr_ref, cnt, nv, lane_sel):
            def _vec(i, carry):
                cv = cid_ref[pl.ds(i * L, L)]
                av = addr_ref[pl.ds(i * L, L)]
                valid = (i * L + iota16) < cnt
                m = (cv == j) & valid & lane_sel
                rows = row_off + (av >> 10)
                cols = av & (N_ROWS - 1)
                if value == 0.0:
                    plsc.store_scatter(buf, [rows, cols], zeros16, mask=m)
                else:
                    plsc.addupdate_scatter(buf, [rows, cols], ones16,
                                           mask=m)
                return carry

            lax.fori_loop(0, nv, _vec, 0)

        true16 = iota16 >= 0
        _one(cidA, addrA, cntA, nvA, true16)
        _one(cidB, addrB, cntB, nvB, true16)

    def _chunk(jj, carry):
        j = wid + NW * jj
        h = (jj & 1) * CH

        @pl.when(jj >= 2)
        def _():
            pltpu.make_async_copy(
                buf.at[pl.ds(h, CH)], out_hbm.at[pl.ds(0, CH)], sem).wait()
            _pass(j - 2 * NW, h, 0.0)

        _pass(j, h, 1.0)
        pltpu.async_copy(
            buf.at[pl.ds(h, CH)], out_hbm.at[pl.ds(j * CH, CH)], sem)
        return carry

    lax.fori_loop(0, QFULL, _chunk, 0)

    pltpu.make_async_copy(
        buf.at[pl.ds(0, CH)], out_hbm.at[pl.ds(0, CH)], sem).wait()
    pltpu.make_async_copy(
        buf.at[pl.ds(0, CH)], out_hbm.at[pl.ds(0, CH)], sem).wait()

    @pl.when(wid < NEXTRA)
    def _():
        j = QFULL * NW + wid
        h = (QFULL & 1) * CH
        _pass(wid + NW * (QFULL - 2), h, 0.0)
        _pass(j, h, 1.0)
        pltpu.sync_copy(buf.at[pl.ds(h, CH)],
                        out_hbm.at[pl.ds(j * CH, CH)])


@jax.jit
def _embed(x_flat):
    mesh = plsc.VectorSubcoreMesh(
        core_axis_name="c", subcore_axis_name="s", num_cores=NC,
        num_subcores=NS)
    f = pl.kernel(
        _body,
        out_type=jax.ShapeDtypeStruct((D_VOCAB, N_ROWS), jnp.float32),
        mesh=mesh,
        scratch_types=[
            pltpu.VMEM((N_TOK,), jnp.int32),
            pltpu.VMEM((LISTCAP,), jnp.int32),
            pltpu.VMEM((LISTCAP,), jnp.int32),
            pltpu.VMEM((LISTCAP,), jnp.int32),
            pltpu.VMEM((LISTCAP,), jnp.int32),
            pltpu.VMEM((2 * CH, N_ROWS), jnp.float32),
            pltpu.SemaphoreType.DMA,
        ],
        compiler_params=pltpu.CompilerParams(needs_layout_passes=False),
    )
    return f(x_flat)


def kernel(x, W_E):
    del W_E
    out_t = _embed(x.reshape(-1).astype(jnp.int32))
    return out_t.T[:, None, :]


# speedup vs baseline: 7.0579x; 1.0836x over previous
import jax
import jax.numpy as jnp
from jax import lax
from jax.experimental import pallas as pl
from jax.experimental.pallas import tpu as pltpu
from jax.experimental.pallas import tpu_sc as plsc

D_VOCAB = 100000
N_ROWS = 1024
N_TOK = 2 * N_ROWS
CH = 32
NCHUNK = D_VOCAB // CH
NC = 2
NS = 16
L = 16
NW = NC * NS
QFULL = NCHUNK // NW
NEXTRA = NCHUNK - QFULL * NW
LISTCAP = N_ROWS + L
RING = 3


def _body(x_hbm, out_hbm, idx_v, cidA, addrA, cidB, addrB, buf, sem):
    wid = lax.axis_index("s") * NC + lax.axis_index("c")

    pltpu.sync_copy(x_hbm, idx_v)

    zeros16 = jnp.zeros((L,), jnp.float32)
    ones16 = jnp.ones((L,), jnp.float32)
    iota16 = lax.iota(jnp.int32, L)
    even16 = (iota16 & 1) == 0
    half16 = iota16 >> 1

    def _zero(i, carry):
        for k in range(N_ROWS // L):
            buf[i, pl.ds(k * L, L)] = zeros16
        return carry

    lax.fori_loop(0, RING * CH, _zero, 0)

    def _scan(q, carry):
        ca, cb = carry
        tvec = idx_v[pl.ds(q * L, L)]
        cid = tvec >> 5
        mine = (cid & (NW - 1)) == wid
        bvec = q * 8 + half16
        addr = ((tvec & (CH - 1)) << 10) | bvec
        mA = mine & even16
        mB = mine & jnp.logical_not(even16)
        plsc.store_compressed(cidA.at[pl.ds(ca, L)], cid, mask=mA)
        plsc.store_compressed(addrA.at[pl.ds(ca, L)], addr, mask=mA)
        plsc.store_compressed(cidB.at[pl.ds(cb, L)], cid, mask=mB)
        plsc.store_compressed(addrB.at[pl.ds(cb, L)], addr, mask=mB)
        ca = ca + jnp.sum(mA.astype(jnp.int32))
        cb = cb + jnp.sum(mB.astype(jnp.int32))
        return ca, cb

    cntA, cntB = lax.fori_loop(0, N_TOK // L, _scan, (jnp.int32(0),
                                                      jnp.int32(0)))
    nvA = (cntA + L - 1) >> 4
    nvB = (cntB + L - 1) >> 4

    def _pass(j, row_off, value):
        def _one(cid_ref, addr_ref, cnt, nv, lane_sel):
            def _vec(i, carry):
                cv = cid_ref[pl.ds(i * L, L)]
                av = addr_ref[pl.ds(i * L, L)]
                valid = (i * L + iota16) < cnt
                m = (cv == j) & valid & lane_sel
                rows = row_off + (av >> 10)
                cols = av & (N_ROWS - 1)
                if value == 0.0:
                    plsc.store_scatter(buf, [rows, cols], zeros16, mask=m)
                else:
                    plsc.addupdate_scatter(buf, [rows, cols], ones16,
                                           mask=m)
                return carry

            lax.fori_loop(0, nv, _vec, 0)

        true16 = iota16 >= 0
        _one(cidA, addrA, cntA, nvA, true16)
        _one(cidB, addrB, cntB, nvB, true16)

    def _chunk(jj, carry):
        j = wid + NW * jj
        h = (jj - (jj // RING) * RING) * CH

        @pl.when(jj >= RING)
        def _():
            pltpu.make_async_copy(
                buf.at[pl.ds(h, CH)], out_hbm.at[pl.ds(0, CH)], sem).wait()
            _pass(j - RING * NW, h, 0.0)

        _pass(j, h, 1.0)
        pltpu.async_copy(
            buf.at[pl.ds(h, CH)], out_hbm.at[pl.ds(j * CH, CH)], sem)
        return carry

    lax.fori_loop(0, QFULL, _chunk, 0)

    for _ in range(RING):
        pltpu.make_async_copy(
            buf.at[pl.ds(0, CH)], out_hbm.at[pl.ds(0, CH)], sem).wait()

    @pl.when(wid < NEXTRA)
    def _():
        j = QFULL * NW + wid
        h = (QFULL % RING) * CH
        _pass(wid + NW * (QFULL - RING), h, 0.0)
        _pass(j, h, 1.0)
        pltpu.sync_copy(buf.at[pl.ds(h, CH)],
                        out_hbm.at[pl.ds(j * CH, CH)])


@jax.jit
def _embed(x_flat):
    mesh = plsc.VectorSubcoreMesh(
        core_axis_name="c", subcore_axis_name="s", num_cores=NC,
        num_subcores=NS)
    f = pl.kernel(
        _body,
        out_type=jax.ShapeDtypeStruct((D_VOCAB, N_ROWS), jnp.float32),
        mesh=mesh,
        scratch_types=[
            pltpu.VMEM((N_TOK,), jnp.int32),
            pltpu.VMEM((LISTCAP,), jnp.int32),
            pltpu.VMEM((LISTCAP,), jnp.int32),
            pltpu.VMEM((LISTCAP,), jnp.int32),
            pltpu.VMEM((LISTCAP,), jnp.int32),
            pltpu.VMEM((RING * CH, N_ROWS), jnp.float32),
            pltpu.SemaphoreType.DMA,
        ],
        compiler_params=pltpu.CompilerParams(needs_layout_passes=False),
    )
    return f(x_flat)


def kernel(x, W_E):
    del W_E
    out_t = _embed(x.reshape(-1).astype(jnp.int32))
    return out_t.T[:, None, :]


# speedup vs baseline: 7.1957x; 1.0195x over previous
import jax
import jax.numpy as jnp
from jax import lax
from jax.experimental import pallas as pl
from jax.experimental.pallas import tpu as pltpu
from jax.experimental.pallas import tpu_sc as plsc

D_VOCAB = 100000
N_ROWS = 1024
N_TOK = 2 * N_ROWS
CH = 32
NCHUNK = D_VOCAB // CH
NC = 2
NS = 16
L = 16
NW = NC * NS
QFULL = NCHUNK // NW
NEXTRA = NCHUNK - QFULL * NW
LISTCAP = N_ROWS + L
RING = 3


def _body(x_hbm, out_hbm, idx_v, cidA, addrA, cidB, addrB, buf, sem):
    wid = lax.axis_index("s") * NC + lax.axis_index("c")

    pltpu.sync_copy(x_hbm, idx_v)

    zeros16 = jnp.zeros((L,), jnp.float32)
    ones16 = jnp.ones((L,), jnp.float32)
    iota16 = lax.iota(jnp.int32, L)
    even16 = (iota16 & 1) == 0
    half16 = iota16 >> 1

    def _scan(q, carry):
        ca, cb = carry
        tvec = idx_v[pl.ds(q * L, L)]
        cid = tvec >> 5
        mine = (cid & (NW - 1)) == wid
        bvec = q * 8 + half16
        addr = ((tvec & (CH - 1)) << 10) | bvec
        mA = mine & even16
        mB = mine & jnp.logical_not(even16)
        plsc.store_compressed(cidA.at[pl.ds(ca, L)], cid, mask=mA)
        plsc.store_compressed(addrA.at[pl.ds(ca, L)], addr, mask=mA)
        plsc.store_compressed(cidB.at[pl.ds(cb, L)], cid, mask=mB)
        plsc.store_compressed(addrB.at[pl.ds(cb, L)], addr, mask=mB)
        ca = ca + jnp.sum(mA.astype(jnp.int32))
        cb = cb + jnp.sum(mB.astype(jnp.int32))
        return ca, cb

    cntA, cntB = lax.fori_loop(0, N_TOK // L, _scan, (jnp.int32(0),
                                                      jnp.int32(0)))
    nvA = (cntA + L - 1) >> 4
    nvB = (cntB + L - 1) >> 4

    def _pass(j, row_off, value):
        def _one(cid_ref, addr_ref, cnt, nv, lane_sel):
            def _vec(i, carry):
                cv = cid_ref[pl.ds(i * L, L)]
                av = addr_ref[pl.ds(i * L, L)]
                valid = (i * L + iota16) < cnt
                m = (cv == j) & valid & lane_sel
                rows = row_off + (av >> 10)
                cols = av & (N_ROWS - 1)
                if value == 0.0:
                    plsc.store_scatter(buf, [rows, cols], zeros16, mask=m)
                else:
                    plsc.addupdate_scatter(buf, [rows, cols], ones16,
                                           mask=m)
                return carry

            lax.fori_loop(0, nv, _vec, 0)

        true16 = iota16 >= 0
        _one(cidA, addrA, cntA, nvA, true16)
        _one(cidB, addrB, cntB, nvB, true16)

    def _chunk(jj, carry):
        j = wid + NW * jj
        h = (jj - (jj // RING) * RING) * CH

        @pl.when(jj < RING)
        def _():
            def _zrow(i, carry):
                for k in range(N_ROWS // L):
                    buf[i, pl.ds(k * L, L)] = zeros16
                return carry

            lax.fori_loop(h, h + CH, _zrow, 0)

        @pl.when(jj >= RING)
        def _():
            pltpu.make_async_copy(
                buf.at[pl.ds(h, CH)], out_hbm.at[pl.ds(0, CH)], sem).wait()
            _pass(j - RING * NW, h, 0.0)

        _pass(j, h, 1.0)
        pltpu.async_copy(
            buf.at[pl.ds(h, CH)], out_hbm.at[pl.ds(j * CH, CH)], sem)
        return carry

    lax.fori_loop(0, QFULL, _chunk, 0)

    for _ in range(RING):
        pltpu.make_async_copy(
            buf.at[pl.ds(0, CH)], out_hbm.at[pl.ds(0, CH)], sem).wait()

    @pl.when(wid < NEXTRA)
    def _():
        j = QFULL * NW + wid
        h = (QFULL % RING) * CH
        _pass(wid + NW * (QFULL - RING), h, 0.0)
        _pass(j, h, 1.0)
        pltpu.sync_copy(buf.at[pl.ds(h, CH)],
                        out_hbm.at[pl.ds(j * CH, CH)])


@jax.jit
def _embed(x_flat):
    mesh = plsc.VectorSubcoreMesh(
        core_axis_name="c", subcore_axis_name="s", num_cores=NC,
        num_subcores=NS)
    f = pl.kernel(
        _body,
        out_type=jax.ShapeDtypeStruct((D_VOCAB, N_ROWS), jnp.float32),
        mesh=mesh,
        scratch_types=[
            pltpu.VMEM((N_TOK,), jnp.int32),
            pltpu.VMEM((LISTCAP,), jnp.int32),
            pltpu.VMEM((LISTCAP,), jnp.int32),
            pltpu.VMEM((LISTCAP,), jnp.int32),
            pltpu.VMEM((LISTCAP,), jnp.int32),
            pltpu.VMEM((RING * CH, N_ROWS), jnp.float32),
            pltpu.SemaphoreType.DMA,
        ],
        compiler_params=pltpu.CompilerParams(needs_layout_passes=False),
    )
    return f(x_flat)


def kernel(x, W_E):
    del W_E
    out_t = _embed(x.reshape(-1).astype(jnp.int32))
    return out_t.T[:, None, :]


# speedup vs baseline: 7.2070x; 1.0016x over previous
import jax
import jax.numpy as jnp
from jax import lax
from jax.experimental import pallas as pl
from jax.experimental.pallas import tpu as pltpu
from jax.experimental.pallas import tpu_sc as plsc

D_VOCAB = 100000
N_ROWS = 1024
N_TOK = 2 * N_ROWS
CH = 16
CH_LOG = 4
NCHUNK = D_VOCAB // CH
NC = 2
NS = 16
L = 16
NW = NC * NS
QFULL = NCHUNK // NW
NEXTRA = NCHUNK - QFULL * NW
LISTCAP = N_ROWS + L
RING = 6


def _body(x_hbm, out_hbm, idx_v, cidA, addrA, cidB, addrB, buf, sem):
    wid = lax.axis_index("s") * NC + lax.axis_index("c")

    pltpu.sync_copy(x_hbm, idx_v)

    zeros16 = jnp.zeros((L,), jnp.float32)
    ones16 = jnp.ones((L,), jnp.float32)
    iota16 = lax.iota(jnp.int32, L)
    even16 = (iota16 & 1) == 0
    half16 = iota16 >> 1

    def _scan(q, carry):
        ca, cb = carry
        tvec = idx_v[pl.ds(q * L, L)]
        cid = tvec >> CH_LOG
        mine = (cid & (NW - 1)) == wid
        bvec = q * 8 + half16
        addr = ((tvec & (CH - 1)) << 10) | bvec
        mA = mine & even16
        mB = mine & jnp.logical_not(even16)
        plsc.store_compressed(cidA.at[pl.ds(ca, L)], cid, mask=mA)
        plsc.store_compressed(addrA.at[pl.ds(ca, L)], addr, mask=mA)
        plsc.store_compressed(cidB.at[pl.ds(cb, L)], cid, mask=mB)
        plsc.store_compressed(addrB.at[pl.ds(cb, L)], addr, mask=mB)
        ca = ca + jnp.sum(mA.astype(jnp.int32))
        cb = cb + jnp.sum(mB.astype(jnp.int32))
        return ca, cb

    cntA, cntB = lax.fori_loop(0, N_TOK // L, _scan, (jnp.int32(0),
                                                      jnp.int32(0)))
    nvA = (cntA + L - 1) >> 4
    nvB = (cntB + L - 1) >> 4

    def _pass(j, row_off, value):
        def _one(cid_ref, addr_ref, cnt, nv, lane_sel):
            def _vec(i, carry):
                cv = cid_ref[pl.ds(i * L, L)]
                av = addr_ref[pl.ds(i * L, L)]
                valid = (i * L + iota16) < cnt
                m = (cv == j) & valid & lane_sel
                rows = row_off + (av >> 10)
                cols = av & (N_ROWS - 1)
                if value == 0.0:
                    plsc.store_scatter(buf, [rows, cols], zeros16, mask=m)
                else:
                    plsc.addupdate_scatter(buf, [rows, cols], ones16,
                                           mask=m)
                return carry

            lax.fori_loop(0, nv, _vec, 0)

        true16 = iota16 >= 0
        _one(cidA, addrA, cntA, nvA, true16)
        _one(cidB, addrB, cntB, nvB, true16)

    def _chunk(jj, carry):
        j = wid + NW * jj
        h = (jj - (jj // RING) * RING) * CH

        @pl.when(jj < RING)
        def _():
            def _zrow(i, carry):
                for k in range(N_ROWS // L):
                    buf[i, pl.ds(k * L, L)] = zeros16
                return carry

            lax.fori_loop(h, h + CH, _zrow, 0)

        @pl.when(jj >= RING)
        def _():
            pltpu.make_async_copy(
                buf.at[pl.ds(h, CH)], out_hbm.at[pl.ds(0, CH)], sem).wait()
            _pass(j - RING * NW, h, 0.0)

        _pass(j, h, 1.0)
        pltpu.async_copy(
            buf.at[pl.ds(h, CH)], out_hbm.at[pl.ds(j * CH, CH)], sem)
        return carry

    lax.fori_loop(0, QFULL, _chunk, 0)

    for _ in range(RING):
        pltpu.make_async_copy(
            buf.at[pl.ds(0, CH)], out_hbm.at[pl.ds(0, CH)], sem).wait()

    @pl.when(wid < NEXTRA)
    def _():
        j = QFULL * NW + wid
        h = (QFULL % RING) * CH
        _pass(wid + NW * (QFULL - RING), h, 0.0)
        _pass(j, h, 1.0)
        pltpu.sync_copy(buf.at[pl.ds(h, CH)],
                        out_hbm.at[pl.ds(j * CH, CH)])


@jax.jit
def _embed(x_flat):
    mesh = plsc.VectorSubcoreMesh(
        core_axis_name="c", subcore_axis_name="s", num_cores=NC,
        num_subcores=NS)
    f = pl.kernel(
        _body,
        out_type=jax.ShapeDtypeStruct((D_VOCAB, N_ROWS), jnp.float32),
        mesh=mesh,
        scratch_types=[
            pltpu.VMEM((N_TOK,), jnp.int32),
            pltpu.VMEM((LISTCAP,), jnp.int32),
            pltpu.VMEM((LISTCAP,), jnp.int32),
            pltpu.VMEM((LISTCAP,), jnp.int32),
            pltpu.VMEM((LISTCAP,), jnp.int32),
            pltpu.VMEM((RING * CH, N_ROWS), jnp.float32),
            pltpu.SemaphoreType.DMA,
        ],
        compiler_params=pltpu.CompilerParams(needs_layout_passes=False),
    )
    return f(x_flat)


def kernel(x, W_E):
    del W_E
    out_t = _embed(x.reshape(-1).astype(jnp.int32))
    return out_t.T[:, None, :]
